# baseline (speedup 1.0000x reference)
import functools

import jax
import jax.numpy as jnp
from jax import lax
from jax.experimental import pallas as pl
from jax.experimental.pallas import tpu as pltpu
from jax.experimental.pallas import tpu_sc as plsc

L_LANES = 16
NC, NS = 2, 16
NW = NC * NS
K = 64


def _splat_sum(v):
    iota = lax.iota(jnp.int32, L_LANES)
    for sh in (8, 4, 2, 1):
        v = v + jnp.take_along_axis(v, iota ^ sh, axis=0)
    return v


def _splat(v, t):
    return jnp.take_along_axis(v, jnp.full((L_LANES,), t, jnp.int32), axis=0)


def _rsqrt(x):
    i = lax.bitcast_convert_type(x, jnp.int32)
    y = lax.bitcast_convert_type(jnp.int32(0x5F3759DF) - (i >> 1), jnp.float32)
    for _ in range(3):
        y = y * (1.5 - 0.5 * x * y * y)
    return y


def _prologue_tc(seg_embed, pos_embed):
    ML, H = pos_embed.shape

    def body(se_ref, pe_ref, sp0_ref, d_ref):
        se = se_ref[...]
        sp0_ref[...] = pe_ref[...] + se[0]
        d_ref[...] = se[1] - se[0]

    return pl.pallas_call(
        body,
        out_shape=(
            jax.ShapeDtypeStruct((ML, H), jnp.float32),
            jax.ShapeDtypeStruct((H,), jnp.float32),
        ),
    )(seg_embed, pos_embed)


def _sc_body(nsteps, H, PB, xw_hbm, sw_hbm, oi_hbm, tok_hbm, sp0_hbm, d_hbm,
             gam_hbm, bet_hbm, out_hbm, xidx, swv, oidx, tokb, base_v, d_v,
             gam_v, bet_v, semt, semw):
    nh = H // L_LANES
    cid = lax.axis_index("c")
    sid = lax.axis_index("s")
    wid = sid * NC + cid

    pltpu.sync_copy(gam_hbm, gam_v)
    pltpu.sync_copy(bet_hbm, bet_v)
    pltpu.sync_copy(d_hbm, d_v)
    pltpu.sync_copy(sp0_hbm.at[pl.ds(wid * PB, PB)], base_v)
    pltpu.sync_copy(xw_hbm.at[wid], xidx)
    pltpu.sync_copy(sw_hbm.at[wid], swv)
    pltpu.sync_copy(oi_hbm.at[wid], oidx)

    def start_gather(i, b):
        pltpu.async_copy(tok_hbm.at[xidx.at[pl.ds(i * K, K)]], tokb[b],
                         semt[b])

    def wait_gather(b):
        pltpu.make_async_copy(tok_hbm.at[xidx.at[pl.ds(0, K)]], tokb[b],
                              semt[b]).wait()

    def start_out(i, b):
        pltpu.async_copy(tokb[b], out_hbm.at[oidx.at[i]], semw[b])

    def wait_out(i, b):
        pltpu.make_async_copy(tokb[b], out_hbm.at[oidx.at[i]], semw[b]).wait()

    def compute(i, b):
        buf = tokb[b]
        inv_h = 1.0 / H
        for g in range(K // L_LANES):
            t0 = g * L_LANES
            ssel = swv[pl.ds(i * K + t0, L_LANES)]

            def pass1(j, acc):
                a1, a2 = acc
                off = j * L_LANES
                rows = [base_v[p, pl.ds(off, L_LANES)] for p in range(PB)]
                dr = d_v[pl.ds(off, L_LANES)]
                n1, n2 = [], []
                for t in range(L_LANES):
                    sp = dr * _splat(ssel, t) + rows[t % PB]
                    e = buf[t0 + t, pl.ds(off, L_LANES)] + sp
                    buf[t0 + t, pl.ds(off, L_LANES)] = e
                    n1.append(a1[t] + e)
                    n2.append(a2[t] + e * e)
                return tuple(n1), tuple(n2)

            zeros = tuple(jnp.zeros((L_LANES,), jnp.float32)
                          for _ in range(L_LANES))
            a1, a2 = lax.fori_loop(0, nh, pass1, (zeros, zeros))

            scale, shift = [], []
            for t in range(L_LANES):
                m = _splat_sum(a1[t]) * inv_h
                var = _splat_sum(a2[t]) * inv_h - m * m
                r = _rsqrt(var + 1e-5)
                scale.append(r)
                shift.append(-m * r)

            def pass2(j, _):
                off = j * L_LANES
                gv = gam_v[pl.ds(off, L_LANES)]
                bv = bet_v[pl.ds(off, L_LANES)]
                for t in range(L_LANES):
                    e = buf[t0 + t, pl.ds(off, L_LANES)]
                    z = e * scale[t] + shift[t]
                    buf[t0 + t, pl.ds(off, L_LANES)] = z * gv + bv
                return 0

            lax.fori_loop(0, nh, pass2, 0)

    start_gather(0, 0)

    def outer(i2, carry):
        for b in (0, 1):
            i = i2 * 2 + b
            nb = 1 - b
            wait_gather(b)

            @pl.when(i > 0)
            def _():
                wait_out(i - 1, nb)

            ip1 = jnp.minimum(i + 1, nsteps - 1)
            start_gather(ip1, nb)
            compute(i, b)
            start_out(i, b)
        return carry

    lax.fori_loop(0, nsteps // 2, outer, 0)

    wait_gather(0)
    wait_out(nsteps - 1, 1)


def kernel(x, seg, tok_embed, seg_embed, pos_embed, ln_gamma, ln_beta):
    B, L = x.shape
    V, H = tok_embed.shape
    N = B * L
    PB = L // NW
    npw = B * PB
    nsteps = npw // K

    sp0, d = _prologue_tc(seg_embed, pos_embed)

    xw = x.reshape(B, NW, PB).swapaxes(0, 1).reshape(NW, npw)
    sw = seg.reshape(B, NW, PB).swapaxes(0, 1).reshape(NW, npw)
    swf = sw.astype(jnp.float32)
    k_ar = jnp.arange(npw, dtype=jnp.int32)
    w_ar = jnp.arange(NW, dtype=jnp.int32)
    oidx = ((k_ar[None, :] // PB) * L + w_ar[:, None] * PB
            + (k_ar[None, :] % PB)).reshape(NW, nsteps, K)

    mesh = plsc.VectorSubcoreMesh(core_axis_name="c", subcore_axis_name="s")
    run = pl.kernel(
        functools.partial(_sc_body, nsteps, H, PB),
        out_type=jax.ShapeDtypeStruct((N, H), jnp.float32),
        mesh=mesh,
        scratch_types=[
            pltpu.VMEM((npw,), jnp.int32),
            pltpu.VMEM((npw,), jnp.float32),
            pltpu.VMEM((nsteps, K), jnp.int32),
            (pltpu.VMEM((K, H), jnp.float32), pltpu.VMEM((K, H), jnp.float32)),
            pltpu.VMEM((PB, H), jnp.float32),
            pltpu.VMEM((H,), jnp.float32),
            pltpu.VMEM((H,), jnp.float32),
            pltpu.VMEM((H,), jnp.float32),
            (pltpu.SemaphoreType.DMA, pltpu.SemaphoreType.DMA),
            (pltpu.SemaphoreType.DMA, pltpu.SemaphoreType.DMA),
        ],
    )
    out = run(xw, swf, oidx, tok_embed, sp0, d, ln_gamma, ln_beta)
    return out.reshape(B, L, H)
